# baseline (speedup 1.0000x reference)
import functools
import jax
import jax.numpy as jnp
from jax import lax
from jax.experimental import pallas as pl
from jax.experimental.pallas import tpu as pltpu
from jax.experimental.pallas import tpu_sc as plsc

_NC = 2
_NS = 16
_NW = _NC * _NS
_CHUNK = 128


def _gather(weight, idx2):
    F, B = idx2.shape
    D = weight.shape[1]
    n_chunks = F
    NBUF = 4
    n_main = n_chunks // NBUF

    mesh = plsc.VectorSubcoreMesh(core_axis_name="c", subcore_axis_name="s")

    @functools.partial(
        pl.kernel,
        mesh=mesh,
        out_type=jax.ShapeDtypeStruct((F * B, D), jnp.float32),
        scratch_types=[
            pltpu.VMEM((F, _CHUNK), jnp.int32),
        ]
        + [pltpu.VMEM((_CHUNK, D), jnp.float32)] * NBUF
        + [pltpu.SemaphoreType.DMA] * (2 * NBUF),
    )
    def k(idx_hbm, table_hbm, out_hbm, idx_v, *bufs_sems):
        bufs = bufs_sems[:NBUF]
        gsem = bufs_sems[NBUF : 2 * NBUF]
        wsem = bufs_sems[2 * NBUF :]
        wid = lax.axis_index("s") * _NC + lax.axis_index("c")
        col0 = wid * _CHUNK
        pltpu.sync_copy(idx_hbm.at[pl.ds(0, F), pl.ds(col0, _CHUNK)], idx_v)

        def g_start(c, bf):
            pltpu.async_copy(table_hbm.at[idx_v.at[c]], bufs[bf], gsem[bf])

        def g_wait(bf):
            pltpu.make_async_copy(table_hbm.at[idx_v.at[0]], bufs[bf], gsem[bf]).wait()

        def w_start(c, bf):
            pltpu.async_copy(
                bufs[bf], out_hbm.at[pl.ds(c * B + col0, _CHUNK)], wsem[bf]
            )

        def w_wait(bf):
            pltpu.make_async_copy(
                bufs[bf], out_hbm.at[pl.ds(col0, _CHUNK)], wsem[bf]
            ).wait()

        for bf in range(NBUF):
            g_start(bf, bf)

        def body(g, carry):
            c0 = g * NBUF
            for bf in range(NBUF):
                g_wait(bf)
                w_start(c0 + bf, bf)
            for bf in range(NBUF):
                nc = c0 + bf + NBUF

                @pl.when(nc < n_chunks)
                def _(nc=nc, bf=bf):
                    w_wait(bf)
                    g_start(nc, bf)

            return carry

        lax.fori_loop(0, n_main, body, 0)
        for bf in range(n_chunks - n_main * NBUF):
            g_wait(bf)
            w_start(n_main * NBUF + bf, bf)
        for bf in range(NBUF):
            w_wait(bf)

    return k(idx2, weight)


def kernel(weight, indices):
    b, f = indices.shape
    d = weight.shape[1]
    idx2 = indices.T.astype(jnp.int32)
    out_flat = _gather(weight, idx2)
    return out_flat.reshape(f, b, d).transpose(1, 0, 2)
